# baseline (speedup 1.0000x reference)
import functools

import jax
import jax.numpy as jnp
from jax.experimental import pallas as pl
from jax.experimental.pallas import tpu as pltpu

BATCH = 1024
INPUT_DIM = 256
NUM_PROTOTYPES = 512
NUM_CLASSES = 8
PER_CLASS = NUM_PROTOTYPES // NUM_CLASSES
LAMBDA_VAL = 1.0


def _glmvq_kernel(x_ref, y_ref, p_ref, omega_ref, out_ref, md_ref, osq_ref):
    c = pl.program_id(0)
    x = x_ref[...]
    om = omega_ref[0]
    osq_c = jnp.sum(om * om).reshape(1, 1)

    @pl.when(c == 0)
    def _():
        osq_ref[...] = osq_c

    @pl.when(c != 0)
    def _():
        osq_ref[...] = osq_ref[...] + osq_c

    tx = jax.lax.dot_general(
        x, om, (((1,), (1,)), ((), ())),
        preferred_element_type=jnp.float32)
    tp = jax.lax.dot_general(
        p_ref[:, 0, 0, :], om, (((1,), (1,)), ((), ())),
        preferred_element_type=jnp.float32)
    norm_tx = jnp.sum(tx * tx, axis=1, keepdims=True)
    tpm2 = -2.0 * tp
    norm_tp = 0.25 * jnp.sum(tpm2 * tpm2, axis=1)
    crossm2 = jax.lax.dot_general(
        tx, tpm2, (((1,), (1,)), ((), ())),
        preferred_element_type=jnp.float32)
    q = crossm2 + norm_tp[None, :]
    mind_c = norm_tx + jnp.min(q, axis=1, keepdims=True)
    lane = jax.lax.broadcasted_iota(jnp.int32, (BATCH, NUM_CLASSES), 1)
    md_ref[...] = jnp.where(lane == c, mind_c, md_ref[...])

    @pl.when(c == NUM_CLASSES - 1)
    def _():
        mt = md_ref[...].T
        y = y_ref[...]
        same = jax.lax.broadcasted_iota(
            jnp.int32, (NUM_CLASSES, BATCH), 0) == y
        inf = jnp.float32(jnp.inf)
        pos = jnp.min(jnp.where(same, mt, inf), axis=0)
        neg = jnp.min(jnp.where(same, inf, mt), axis=0)
        mu = (pos - neg) / (pos + neg)
        loss = jnp.mean(1.0 / (1.0 + jnp.exp(-LAMBDA_VAL * mu)))
        out_ref[...] = (loss + 0.01 * jnp.sqrt(osq_ref[0, 0])).reshape(1, 1)


@functools.partial(jax.jit, static_argnames=())
def kernel(x, y, prototypes, omega):
    protos_r = prototypes.reshape(PER_CLASS, NUM_CLASSES, 1, INPUT_DIM)
    y2 = y.reshape(1, BATCH)
    out = pl.pallas_call(
        _glmvq_kernel,
        grid=(NUM_CLASSES,),
        out_shape=jax.ShapeDtypeStruct((1, 1), jnp.float32),
        in_specs=[
            pl.BlockSpec(memory_space=pltpu.MemorySpace.VMEM),
            pl.BlockSpec(memory_space=pltpu.MemorySpace.VMEM),
            pl.BlockSpec((PER_CLASS, 1, 1, INPUT_DIM), lambda c: (0, c, 0, 0)),
            pl.BlockSpec((1, INPUT_DIM, INPUT_DIM), lambda c: (c, 0, 0)),
        ],
        out_specs=pl.BlockSpec((1, 1), lambda c: (0, 0)),
        scratch_shapes=[
            pltpu.VMEM((BATCH, NUM_CLASSES), jnp.float32),
            pltpu.VMEM((1, 1), jnp.float32),
        ],
        compiler_params=pltpu.CompilerParams(
            dimension_semantics=("arbitrary",)),
    )(x, y2, protos_r, omega)
    return out[0, 0]


# speedup vs baseline: 1.4513x; 1.4513x over previous
import functools

import jax
import jax.numpy as jnp
from jax.experimental import pallas as pl
from jax.experimental.pallas import tpu as pltpu

BATCH = 1024
INPUT_DIM = 256
NUM_PROTOTYPES = 512
NUM_CLASSES = 8
PER_CLASS = NUM_PROTOTYPES // NUM_CLASSES
LAMBDA_VAL = 1.0


def _glmvq_kernel(x_ref, y_ref, p_ref, omega_ref, out_ref):
    x = x_ref[...]
    ntx_cols = []
    minq_rows = []
    omr = omega_ref[...].reshape(NUM_CLASSES * INPUT_DIM, INPUT_DIM)
    omega_sq = jnp.sum(omr * omr)
    for c in range(NUM_CLASSES):
        om = omega_ref[c]
        tx = jax.lax.dot_general(
            x, om, (((1,), (1,)), ((), ())),
            preferred_element_type=jnp.float32)
        tp = jax.lax.dot_general(
            p_ref[:, c * INPUT_DIM:(c + 1) * INPUT_DIM], om,
            (((1,), (1,)), ((), ())),
            preferred_element_type=jnp.float32)
        ntx_cols.append(jnp.sum(tx * tx, axis=1, keepdims=True))
        tpm2 = -2.0 * tp
        norm_tp = 0.25 * jnp.sum(tpm2 * tpm2, axis=1, keepdims=True)
        crossm2 = jax.lax.dot_general(
            tpm2, tx, (((1,), (1,)), ((), ())),
            preferred_element_type=jnp.float32)
        q = crossm2 + norm_tp
        minq_rows.append(jnp.min(q, axis=0, keepdims=True))
    ntx = jnp.concatenate(ntx_cols, axis=1)
    minq = jnp.concatenate(minq_rows, axis=0)
    mt = ntx.T + minq
    y = y_ref[...]
    same = jax.lax.broadcasted_iota(jnp.int32, (NUM_CLASSES, BATCH), 0) == y
    inf = jnp.float32(jnp.inf)
    pos = jnp.min(jnp.where(same, mt, inf), axis=0)
    neg = jnp.min(jnp.where(same, inf, mt), axis=0)
    mu = (pos - neg) / (pos + neg)
    loss = jnp.mean(1.0 / (1.0 + jnp.exp(-LAMBDA_VAL * mu)))
    out_ref[...] = (loss + 0.01 * jnp.sqrt(omega_sq)).reshape(1, 1)


@functools.partial(jax.jit, static_argnames=())
def kernel(x, y, prototypes, omega):
    protos_r = prototypes.reshape(PER_CLASS, NUM_CLASSES * INPUT_DIM)
    y2 = y.reshape(1, BATCH)
    out = pl.pallas_call(
        _glmvq_kernel,
        out_shape=jax.ShapeDtypeStruct((1, 1), jnp.float32),
    )(x, y2, protos_r, omega)
    return out[0, 0]


# speedup vs baseline: 1.4787x; 1.0189x over previous
import functools

import jax
import jax.numpy as jnp
from jax.experimental import pallas as pl
from jax.experimental.pallas import tpu as pltpu

BATCH = 1024
INPUT_DIM = 256
NUM_PROTOTYPES = 512
NUM_CLASSES = 8
PER_CLASS = NUM_PROTOTYPES // NUM_CLASSES
LAMBDA_VAL = 1.0


def _glmvq_kernel(x_ref, y_ref, p_ref, omega_ref, out_ref):
    x = x_ref[...]
    ntx_cols = []
    minq_rows = []
    omr = omega_ref[...].reshape(NUM_CLASSES * INPUT_DIM, INPUT_DIM)
    omega_sq = jnp.sum(omr * omr)
    tx_all = jax.lax.dot_general(
        x, omr, (((1,), (1,)), ((), ())),
        preferred_element_type=jnp.float32)
    for c in range(NUM_CLASSES):
        om = omega_ref[c]
        tx = tx_all[:, c * INPUT_DIM:(c + 1) * INPUT_DIM]
        tp = jax.lax.dot_general(
            p_ref[:, c * INPUT_DIM:(c + 1) * INPUT_DIM], om,
            (((1,), (1,)), ((), ())),
            preferred_element_type=jnp.float32)
        ntx_cols.append(jnp.sum(tx * tx, axis=1, keepdims=True))
        tpm2 = -2.0 * tp
        norm_tp = 0.25 * jnp.sum(tpm2 * tpm2, axis=1, keepdims=True)
        crossm2 = jax.lax.dot_general(
            tpm2, tx, (((1,), (1,)), ((), ())),
            preferred_element_type=jnp.float32)
        q = crossm2 + norm_tp
        minq_rows.append(jnp.min(q, axis=0, keepdims=True))
    ntx = jnp.concatenate(ntx_cols, axis=1)
    minq = jnp.concatenate(minq_rows, axis=0)
    mt = ntx.T + minq
    y = y_ref[...]
    same = jax.lax.broadcasted_iota(jnp.int32, (NUM_CLASSES, BATCH), 0) == y
    inf = jnp.float32(jnp.inf)
    pos = jnp.min(jnp.where(same, mt, inf), axis=0)
    neg = jnp.min(jnp.where(same, inf, mt), axis=0)
    mu = (pos - neg) / (pos + neg)
    loss = jnp.mean(1.0 / (1.0 + jnp.exp(-LAMBDA_VAL * mu)))
    out_ref[...] = (loss + 0.01 * jnp.sqrt(omega_sq)).reshape(1, 1)


@functools.partial(jax.jit, static_argnames=())
def kernel(x, y, prototypes, omega):
    protos_r = prototypes.reshape(PER_CLASS, NUM_CLASSES * INPUT_DIM)
    y2 = y.reshape(1, BATCH)
    out = pl.pallas_call(
        _glmvq_kernel,
        out_shape=jax.ShapeDtypeStruct((1, 1), jnp.float32),
    )(x, y2, protos_r, omega)
    return out[0, 0]


# speedup vs baseline: 1.4853x; 1.0045x over previous
import functools

import jax
import jax.numpy as jnp
from jax.experimental import pallas as pl
from jax.experimental.pallas import tpu as pltpu

BATCH = 1024
INPUT_DIM = 256
NUM_PROTOTYPES = 512
NUM_CLASSES = 8
PER_CLASS = NUM_PROTOTYPES // NUM_CLASSES
LAMBDA_VAL = 1.0


def _glmvq_kernel(x_ref, y_ref, p_ref, omega_ref, out_ref):
    x = x_ref[...]
    ntx_cols = []
    minq_rows = []
    omr = omega_ref[...].reshape(NUM_CLASSES * INPUT_DIM, INPUT_DIM)
    omega_sq = jnp.sum(omr * omr)
    tx_all = jax.lax.dot_general(
        x, omr, (((1,), (1,)), ((), ())),
        preferred_element_type=jnp.float32)
    for c in range(NUM_CLASSES):
        om = omr[c * INPUT_DIM:(c + 1) * INPUT_DIM, :]
        tx = tx_all[:, c * INPUT_DIM:(c + 1) * INPUT_DIM]
        tp = jax.lax.dot_general(
            p_ref[:, c * INPUT_DIM:(c + 1) * INPUT_DIM], om,
            (((1,), (1,)), ((), ())),
            preferred_element_type=jnp.float32)
        ntx_cols.append(jnp.sum(tx * tx, axis=1, keepdims=True))
        tpm2 = -2.0 * tp
        norm_tp = 0.25 * jnp.sum(tpm2 * tpm2, axis=1, keepdims=True)
        crossm2 = jax.lax.dot_general(
            tpm2, tx, (((1,), (1,)), ((), ())),
            preferred_element_type=jnp.float32)
        q = crossm2 + norm_tp
        minq_rows.append(jnp.min(q, axis=0, keepdims=True))
    ntx = jnp.concatenate(ntx_cols, axis=1)
    minq = jnp.concatenate(minq_rows, axis=0)
    mt = ntx.T + minq
    y = y_ref[...]
    same = jax.lax.broadcasted_iota(jnp.int32, (NUM_CLASSES, BATCH), 0) == y
    inf = jnp.float32(jnp.inf)
    pos = jnp.min(jnp.where(same, mt, inf), axis=0)
    neg = jnp.min(jnp.where(same, inf, mt), axis=0)
    mu = (pos - neg) / (pos + neg)
    loss = jnp.mean(1.0 / (1.0 + jnp.exp(-LAMBDA_VAL * mu)))
    out_ref[...] = (loss + 0.01 * jnp.sqrt(omega_sq)).reshape(1, 1)


@functools.partial(jax.jit, static_argnames=())
def kernel(x, y, prototypes, omega):
    protos_r = prototypes.reshape(PER_CLASS, NUM_CLASSES * INPUT_DIM)
    y2 = y.reshape(1, BATCH)
    out = pl.pallas_call(
        _glmvq_kernel,
        out_shape=jax.ShapeDtypeStruct((1, 1), jnp.float32),
    )(x, y2, protos_r, omega)
    return out[0, 0]


# speedup vs baseline: 1.4885x; 1.0021x over previous
import functools

import jax
import jax.numpy as jnp
from jax.experimental import pallas as pl

BATCH = 1024
INPUT_DIM = 256
NUM_PROTOTYPES = 512
NUM_CLASSES = 8
PER_CLASS = NUM_PROTOTYPES // NUM_CLASSES
LAMBDA_VAL = 1.0


def _glmvq_kernel(x_ref, y_ref, p_ref, omega_ref, out_ref):
    x = x_ref[...]
    ntx_cols = []
    minq_rows = []
    omr = omega_ref[...].reshape(NUM_CLASSES * INPUT_DIM, INPUT_DIM)
    omega_sq = jnp.sum(omr * omr)
    tx_all = jax.lax.dot_general(
        x, omr, (((1,), (1,)), ((), ())),
        preferred_element_type=jnp.float32)
    for c in range(NUM_CLASSES):
        om = omr[c * INPUT_DIM:(c + 1) * INPUT_DIM, :]
        tx = tx_all[:, c * INPUT_DIM:(c + 1) * INPUT_DIM]
        tp = jax.lax.dot_general(
            p_ref[:, c * INPUT_DIM:(c + 1) * INPUT_DIM], om,
            (((1,), (1,)), ((), ())),
            preferred_element_type=jnp.float32)
        ntx_cols.append(jnp.sum(tx * tx, axis=1, keepdims=True))
        tpm2 = -2.0 * tp
        norm_tp = 0.25 * jnp.sum(tpm2 * tpm2, axis=1, keepdims=True)
        crossm2 = jax.lax.dot_general(
            tpm2, tx, (((1,), (1,)), ((), ())),
            preferred_element_type=jnp.float32)
        q = crossm2 + norm_tp
        minq_rows.append(jnp.min(q, axis=0, keepdims=True))
    ntx = jnp.concatenate(ntx_cols, axis=1)
    minq = jnp.concatenate(minq_rows, axis=0)
    mt = ntx.T + minq
    y = y_ref[...]
    same = jax.lax.broadcasted_iota(jnp.int32, (NUM_CLASSES, BATCH), 0) == y
    inf = jnp.float32(jnp.inf)
    pos = jnp.min(jnp.where(same, mt, inf), axis=0)
    neg = jnp.min(jnp.where(same, inf, mt), axis=0)
    mu = (pos - neg) / (pos + neg)
    loss = jnp.mean(1.0 / (1.0 + jnp.exp(-LAMBDA_VAL * mu)))
    out_ref[...] = (loss + 0.01 * jnp.sqrt(omega_sq)).reshape(1, 1)


@functools.partial(jax.jit, static_argnames=())
def kernel(x, y, prototypes, omega):
    protos_r = prototypes.reshape(PER_CLASS, NUM_CLASSES * INPUT_DIM)
    y2 = y.reshape(1, BATCH)
    out = pl.pallas_call(
        _glmvq_kernel,
        out_shape=jax.ShapeDtypeStruct((1, 1), jnp.float32),
    )(x, y2, protos_r, omega)
    return out[0, 0]


# speedup vs baseline: 1.6302x; 1.0952x over previous
import functools

import jax
import jax.numpy as jnp
from jax.experimental import pallas as pl

BATCH = 1024
INPUT_DIM = 256
NUM_PROTOTYPES = 512
NUM_CLASSES = 8
PER_CLASS = NUM_PROTOTYPES // NUM_CLASSES
LAMBDA_VAL = 1.0


def _glmvq_kernel(x_ref, y_ref, p_ref, omega_ref, out_ref):
    x = x_ref[...]
    ntx_cols = []
    minq_rows = []
    omr = omega_ref[...].reshape(NUM_CLASSES * INPUT_DIM, INPUT_DIM)
    omega_sq = jnp.sum(omr * omr)
    tpm2s, norm_tps = [], []
    for c in range(NUM_CLASSES):
        om = omr[c * INPUT_DIM:(c + 1) * INPUT_DIM, :]
        tp = jax.lax.dot_general(
            p_ref[:, c * INPUT_DIM:(c + 1) * INPUT_DIM], om,
            (((1,), (1,)), ((), ())),
            preferred_element_type=jnp.float32)
        tpm2 = -2.0 * tp
        tpm2s.append(tpm2)
        norm_tps.append(0.25 * jnp.sum(tpm2 * tpm2, axis=1, keepdims=True))
    tx_all = jax.lax.dot_general(
        x, omr, (((1,), (1,)), ((), ())),
        preferred_element_type=jnp.float32)
    for c in range(NUM_CLASSES):
        tx = tx_all[:, c * INPUT_DIM:(c + 1) * INPUT_DIM]
        ntx_cols.append(jnp.sum(tx * tx, axis=1, keepdims=True))
        crossm2 = jax.lax.dot_general(
            tpm2s[c], tx, (((1,), (1,)), ((), ())),
            preferred_element_type=jnp.float32)
        q = crossm2 + norm_tps[c]
        minq_rows.append(jnp.min(q, axis=0, keepdims=True))
    ntx = jnp.concatenate(ntx_cols, axis=1)
    minq = jnp.concatenate(minq_rows, axis=0)
    mt = ntx.T + minq
    y = y_ref[...]
    same = jax.lax.broadcasted_iota(jnp.int32, (NUM_CLASSES, BATCH), 0) == y
    inf = jnp.float32(jnp.inf)
    pos = jnp.min(jnp.where(same, mt, inf), axis=0)
    neg = jnp.min(jnp.where(same, inf, mt), axis=0)
    mu = (pos - neg) / (pos + neg)
    loss = jnp.mean(1.0 / (1.0 + jnp.exp(-LAMBDA_VAL * mu)))
    out_ref[...] = (loss + 0.01 * jnp.sqrt(omega_sq)).reshape(1, 1)


@functools.partial(jax.jit, static_argnames=())
def kernel(x, y, prototypes, omega):
    protos_r = prototypes.reshape(PER_CLASS, NUM_CLASSES * INPUT_DIM)
    y2 = y.reshape(1, BATCH)
    out = pl.pallas_call(
        _glmvq_kernel,
        out_shape=jax.ShapeDtypeStruct((1, 1), jnp.float32),
    )(x, y2, protos_r, omega)
    return out[0, 0]
